# baseline (speedup 1.0000x reference)
import jax
import jax.numpy as jnp
from jax import lax
from jax.experimental import pallas as pl
from jax.experimental.pallas import tpu as pltpu
from jax.experimental.pallas import tpu_sc as plsc

NUM_ATOMS = 100000
ACC_ROWS = 102400
NC = 2
NS = 16
NW = NC * NS
L = 16
B = 2000
ROW_PAD = 8
ROWS_PER_TILE = ACC_ROWS // NS
CCH = 1600


def _cv_kernel(x_hbm, y_hbm, z_hbm, idx_hbm, zero_hbm, out_hbm,
               xa, ya, za, ia, va, xb, yb, zb, ib, vb, abuf, cbuf, acc,
               sem_in_a, sem_in_b, sem_add_a, sem_add_b):
    cid = lax.axis_index("c")
    sid = lax.axis_index("s")
    wid = cid * NS + sid

    n_edges = idx_hbm.shape[0]
    per_tile = n_edges // NW
    n2 = per_tile // (2 * B)

    pltpu.sync_copy(zero_hbm, acc.at[pl.ds(sid * ROWS_PER_TILE, ROWS_PER_TILE)])
    pltpu.sync_copy(zero_hbm.at[pl.ds(0, B)], va)
    pltpu.sync_copy(zero_hbm.at[pl.ds(0, B)], vb)
    plsc.subcore_barrier()

    iota = lax.iota(jnp.int32, L)
    col0 = iota * 0
    col1 = col0 + 1
    magic = jnp.int32(0x5F3759DF)
    c_half = jnp.float32(0.5)
    c_3half = jnp.float32(1.5)

    def start_in(k, xr, yr, zr, ir, sem):
        e0 = wid * per_tile + k * B
        pltpu.make_async_copy(x_hbm.at[pl.ds(e0, B)], xr, sem).start()
        pltpu.make_async_copy(y_hbm.at[pl.ds(e0, B)], yr, sem).start()
        pltpu.make_async_copy(z_hbm.at[pl.ds(e0, B)], zr, sem).start()
        pltpu.make_async_copy(idx_hbm.at[pl.ds(e0, B)], ir, sem).start()

    def wait_in(xr, yr, zr, ir, sem):
        pltpu.make_async_copy(x_hbm.at[pl.ds(0, B)], xr, sem).wait()
        pltpu.make_async_copy(y_hbm.at[pl.ds(0, B)], yr, sem).wait()
        pltpu.make_async_copy(z_hbm.at[pl.ds(0, B)], zr, sem).wait()
        pltpu.make_async_copy(idx_hbm.at[pl.ds(0, B)], ir, sem).wait()

    def compute(xr, yr, zr, vr):
        @plsc.parallel_loop(0, B // L, unroll=4)
        def _(j):
            sl = pl.ds(j * L, L)
            ex = xr[sl]
            ey = yr[sl]
            ez = zr[sl]
            s = ex * ex + ey * ey + ez * ez
            half_s = s * c_half
            y = plsc.bitcast(magic - (plsc.bitcast(s, jnp.int32) >> 1),
                             jnp.float32)
            y = y * (c_3half - half_s * y * y)
            y = y * (c_3half - half_s * y * y)
            y = y * (c_3half - half_s * y * y)
            rows16 = j * L + iota
            plsc.store_scatter(vr, [rows16, col0], y)
            plsc.store_scatter(vr, [rows16, col1], y * y)

    def start_add(vr, ir, sem):
        pltpu.make_async_copy(vr, acc.at[ir], sem).start(add=True)

    def wait_add(vr, ir, sem):
        pltpu.make_async_copy(vr, acc.at[ir], sem).wait()

    start_in(0, xa, ya, za, ia, sem_in_a)

    def body(k, carry):
        @pl.when(k > 0)
        def _():
            wait_add(vb, ib, sem_add_b)
        start_in(2 * k + 1, xb, yb, zb, ib, sem_in_b)
        wait_in(xa, ya, za, ia, sem_in_a)
        compute(xa, ya, za, va)
        start_add(va, ia, sem_add_a)
        wait_in(xb, yb, zb, ib, sem_in_b)
        compute(xb, yb, zb, vb)
        wait_add(va, ia, sem_add_a)

        @pl.when(k < n2 - 1)
        def _():
            start_in(2 * k + 2, xa, ya, za, ia, sem_in_a)
        start_add(vb, ib, sem_add_b)
        return carry

    lax.fori_loop(0, n2, body, None)
    wait_add(vb, ib, sem_add_b)

    plsc.subcore_barrier()

    for c in range(ROWS_PER_TILE // CCH):
        r0 = sid * ROWS_PER_TILE + c * CCH
        pltpu.sync_copy(acc.at[pl.ds(r0, CCH)], abuf)

        @plsc.parallel_loop(0, CCH // L, unroll=4)
        def _(j):
            rr = j * L + iota
            plsc.store_scatter(cbuf, [rr, col0],
                               plsc.load_gather(abuf, [rr, col0]))
            plsc.store_scatter(cbuf, [rr, col1],
                               plsc.load_gather(abuf, [rr, col1]))

        pltpu.sync_copy(cbuf, out_hbm.at[cid].at[pl.ds(r0, CCH)])


def kernel(neighbor_vectors, first_atom, n_atoms):
    del n_atoms
    n_edges = first_atom.shape[0]
    assert n_edges % (NW * 2 * B) == 0

    xs = neighbor_vectors[:, 0]
    ys = neighbor_vectors[:, 1]
    zs = neighbor_vectors[:, 2]
    zero_rows = jnp.zeros((ROWS_PER_TILE, ROW_PAD), jnp.float32)

    mesh = plsc.VectorSubcoreMesh(
        core_axis_name="c", subcore_axis_name="s", num_cores=NC,
        num_subcores=NS)
    partial = pl.kernel(
        _cv_kernel,
        out_type=jax.ShapeDtypeStruct((NC, ACC_ROWS, 2), jnp.float32),
        mesh=mesh,
        scratch_types=[
            pltpu.VMEM((B,), jnp.float32),
            pltpu.VMEM((B,), jnp.float32),
            pltpu.VMEM((B,), jnp.float32),
            pltpu.VMEM((B,), jnp.int32),
            pltpu.VMEM((B, ROW_PAD), jnp.float32),
            pltpu.VMEM((B,), jnp.float32),
            pltpu.VMEM((B,), jnp.float32),
            pltpu.VMEM((B,), jnp.float32),
            pltpu.VMEM((B,), jnp.int32),
            pltpu.VMEM((B, ROW_PAD), jnp.float32),
            pltpu.VMEM((CCH, ROW_PAD), jnp.float32),
            pltpu.VMEM((CCH, 2), jnp.float32),
            pltpu.VMEM_SHARED((ACC_ROWS, ROW_PAD), jnp.float32),
            pltpu.SemaphoreType.DMA,
            pltpu.SemaphoreType.DMA,
            pltpu.SemaphoreType.DMA,
            pltpu.SemaphoreType.DMA,
        ],
        compiler_params=pltpu.CompilerParams(
            needs_layout_passes=False, use_tc_tiling_on_sc=False),
    )(xs, ys, zs, first_atom, zero_rows)
    return (partial[0] + partial[1])[:NUM_ATOMS]


# speedup vs baseline: 1.0731x; 1.0731x over previous
import jax
import jax.numpy as jnp
from jax import lax
from jax.experimental import pallas as pl
from jax.experimental.pallas import tpu as pltpu
from jax.experimental.pallas import tpu_sc as plsc

NUM_ATOMS = 100000
ACC_ROWS = 102400
NC = 2
NS = 16
NW = NC * NS
L = 16
B = 2000
ROW_PAD = 8
ROWS_PER_TILE = ACC_ROWS // NS
CCH = 1600


def _cv_kernel(x_hbm, y_hbm, z_hbm, idx_hbm, zero_hbm, out_hbm,
               xa, ya, za, ia, va, xb, yb, zb, ib, vb, acc,
               sem_in_a, sem_in_b, sem_add_a, sem_add_b):
    cid = lax.axis_index("c")
    sid = lax.axis_index("s")
    wid = cid * NS + sid

    n_edges = idx_hbm.shape[0]
    per_tile = n_edges // NW
    n2 = per_tile // (2 * B)

    pltpu.sync_copy(zero_hbm, acc.at[pl.ds(sid * ROWS_PER_TILE, ROWS_PER_TILE)])
    pltpu.sync_copy(zero_hbm.at[pl.ds(0, B)], va)
    pltpu.sync_copy(zero_hbm.at[pl.ds(0, B)], vb)
    plsc.subcore_barrier()

    iota = lax.iota(jnp.int32, L)
    col0 = iota * 0
    col1 = col0 + 1
    magic = jnp.int32(0x5F3759DF)
    c_half = jnp.float32(0.5)
    c_3half = jnp.float32(1.5)

    def start_in(k, xr, yr, zr, ir, sem):
        e0 = wid * per_tile + k * B
        pltpu.make_async_copy(x_hbm.at[pl.ds(e0, B)], xr, sem).start()
        pltpu.make_async_copy(y_hbm.at[pl.ds(e0, B)], yr, sem).start()
        pltpu.make_async_copy(z_hbm.at[pl.ds(e0, B)], zr, sem).start()
        pltpu.make_async_copy(idx_hbm.at[pl.ds(e0, B)], ir, sem).start()

    def wait_in(xr, yr, zr, ir, sem):
        pltpu.make_async_copy(x_hbm.at[pl.ds(0, B)], xr, sem).wait()
        pltpu.make_async_copy(y_hbm.at[pl.ds(0, B)], yr, sem).wait()
        pltpu.make_async_copy(z_hbm.at[pl.ds(0, B)], zr, sem).wait()
        pltpu.make_async_copy(idx_hbm.at[pl.ds(0, B)], ir, sem).wait()

    def compute(xr, yr, zr, vr):
        @plsc.parallel_loop(0, B // L, unroll=4)
        def _(j):
            sl = pl.ds(j * L, L)
            ex = xr[sl]
            ey = yr[sl]
            ez = zr[sl]
            s = ex * ex + ey * ey + ez * ez
            half_s = s * c_half
            y = plsc.bitcast(magic - (plsc.bitcast(s, jnp.int32) >> 1),
                             jnp.float32)
            y = y * (c_3half - half_s * y * y)
            y = y * (c_3half - half_s * y * y)
            y = y * (c_3half - half_s * y * y)
            rows16 = j * L + iota
            plsc.store_scatter(vr, [rows16, col0], y)
            plsc.store_scatter(vr, [rows16, col1], y * y)

    def start_add(vr, ir, sem):
        pltpu.make_async_copy(vr, acc.at[ir], sem).start(add=True)

    def wait_add(vr, ir, sem):
        pltpu.make_async_copy(vr, acc.at[ir], sem).wait()

    start_in(0, xa, ya, za, ia, sem_in_a)

    def body(k, carry):
        @pl.when(k > 0)
        def _():
            wait_add(vb, ib, sem_add_b)
        start_in(2 * k + 1, xb, yb, zb, ib, sem_in_b)
        wait_in(xa, ya, za, ia, sem_in_a)
        compute(xa, ya, za, va)
        start_add(va, ia, sem_add_a)
        wait_in(xb, yb, zb, ib, sem_in_b)
        compute(xb, yb, zb, vb)
        wait_add(va, ia, sem_add_a)

        @pl.when(k < n2 - 1)
        def _():
            start_in(2 * k + 2, xa, ya, za, ia, sem_in_a)
        start_add(vb, ib, sem_add_b)
        return carry

    lax.fori_loop(0, n2, body, None)
    wait_add(vb, ib, sem_add_b)

    plsc.subcore_barrier()

    @pl.when(sid == 0)
    def _():
        pltpu.sync_copy(acc, out_hbm.at[cid])


def kernel(neighbor_vectors, first_atom, n_atoms):
    del n_atoms
    n_edges = first_atom.shape[0]
    assert n_edges % (NW * 2 * B) == 0

    xs = neighbor_vectors[:, 0]
    ys = neighbor_vectors[:, 1]
    zs = neighbor_vectors[:, 2]
    zero_rows = jnp.zeros((ROWS_PER_TILE, ROW_PAD), jnp.float32)

    mesh = plsc.VectorSubcoreMesh(
        core_axis_name="c", subcore_axis_name="s", num_cores=NC,
        num_subcores=NS)
    partial = pl.kernel(
        _cv_kernel,
        out_type=jax.ShapeDtypeStruct((NC, ACC_ROWS, ROW_PAD), jnp.float32),
        mesh=mesh,
        scratch_types=[
            pltpu.VMEM((B,), jnp.float32),
            pltpu.VMEM((B,), jnp.float32),
            pltpu.VMEM((B,), jnp.float32),
            pltpu.VMEM((B,), jnp.int32),
            pltpu.VMEM((B, ROW_PAD), jnp.float32),
            pltpu.VMEM((B,), jnp.float32),
            pltpu.VMEM((B,), jnp.float32),
            pltpu.VMEM((B,), jnp.float32),
            pltpu.VMEM((B,), jnp.int32),
            pltpu.VMEM((B, ROW_PAD), jnp.float32),
            pltpu.VMEM_SHARED((ACC_ROWS, ROW_PAD), jnp.float32),
            pltpu.SemaphoreType.DMA,
            pltpu.SemaphoreType.DMA,
            pltpu.SemaphoreType.DMA,
            pltpu.SemaphoreType.DMA,
        ],
        compiler_params=pltpu.CompilerParams(
            needs_layout_passes=False, use_tc_tiling_on_sc=False),
    )(xs, ys, zs, first_atom, zero_rows)
    return (partial[0] + partial[1])[:NUM_ATOMS, :2]


# speedup vs baseline: 1.1648x; 1.0855x over previous
import jax
import jax.numpy as jnp
from jax import lax
from jax.experimental import pallas as pl
from jax.experimental.pallas import tpu as pltpu
from jax.experimental.pallas import tpu_sc as plsc

NUM_ATOMS = 100000
ACC_ROWS = NUM_ATOMS
NC = 2
NS = 16
NW = NC * NS
L = 16
B = 2000
ROW_PAD = 8
ROWS_PER_TILE = ACC_ROWS // NS


def _cv_kernel(x_hbm, y_hbm, z_hbm, idx_hbm, zero_hbm, out_hbm,
               xa, ya, za, ia, va, xb, yb, zb, ib, vb, acc,
               sem_in_a, sem_in_b, sem_add_a, sem_add_b):
    cid = lax.axis_index("c")
    sid = lax.axis_index("s")
    wid = cid * NS + sid

    n_edges = idx_hbm.shape[0]
    per_tile = n_edges // NW
    n2 = per_tile // (2 * B)

    pltpu.sync_copy(zero_hbm, acc.at[pl.ds(sid * ROWS_PER_TILE, ROWS_PER_TILE)])
    pltpu.sync_copy(zero_hbm.at[pl.ds(0, B)], va)
    pltpu.sync_copy(zero_hbm.at[pl.ds(0, B)], vb)
    plsc.subcore_barrier()

    iota = lax.iota(jnp.int32, L)
    col0 = iota * 0
    col1 = col0 + 1
    magic = jnp.int32(0x5F3759DF)
    c_half = jnp.float32(0.5)
    c_3half = jnp.float32(1.5)

    def start_in(k, xr, yr, zr, ir, sem):
        e0 = wid * per_tile + k * B
        pltpu.make_async_copy(x_hbm.at[pl.ds(e0, B)], xr, sem).start()
        pltpu.make_async_copy(y_hbm.at[pl.ds(e0, B)], yr, sem).start()
        pltpu.make_async_copy(z_hbm.at[pl.ds(e0, B)], zr, sem).start()
        pltpu.make_async_copy(idx_hbm.at[pl.ds(e0, B)], ir, sem).start()

    def wait_in(xr, yr, zr, ir, sem):
        pltpu.make_async_copy(x_hbm.at[pl.ds(0, B)], xr, sem).wait()
        pltpu.make_async_copy(y_hbm.at[pl.ds(0, B)], yr, sem).wait()
        pltpu.make_async_copy(z_hbm.at[pl.ds(0, B)], zr, sem).wait()
        pltpu.make_async_copy(idx_hbm.at[pl.ds(0, B)], ir, sem).wait()

    def compute(xr, yr, zr, vr):
        @plsc.parallel_loop(0, B // L, unroll=4)
        def _(j):
            sl = pl.ds(j * L, L)
            ex = xr[sl]
            ey = yr[sl]
            ez = zr[sl]
            s = ex * ex + ey * ey + ez * ez
            half_s = s * c_half
            y = plsc.bitcast(magic - (plsc.bitcast(s, jnp.int32) >> 1),
                             jnp.float32)
            y = y * (c_3half - half_s * y * y)
            y = y * (c_3half - half_s * y * y)
            y = y * (c_3half - half_s * y * y)
            rows16 = j * L + iota
            plsc.store_scatter(vr, [rows16, col0], y)
            plsc.store_scatter(vr, [rows16, col1], y * y)

    def start_add(vr, ir, sem):
        pltpu.make_async_copy(vr, acc.at[ir], sem).start(add=True)

    def wait_add(vr, ir, sem):
        pltpu.make_async_copy(vr, acc.at[ir], sem).wait()

    start_in(0, xa, ya, za, ia, sem_in_a)

    def body(k, carry):
        @pl.when(k > 0)
        def _():
            wait_add(vb, ib, sem_add_b)
        start_in(2 * k + 1, xb, yb, zb, ib, sem_in_b)
        wait_in(xa, ya, za, ia, sem_in_a)
        compute(xa, ya, za, va)
        start_add(va, ia, sem_add_a)
        wait_in(xb, yb, zb, ib, sem_in_b)
        compute(xb, yb, zb, vb)
        wait_add(va, ia, sem_add_a)

        @pl.when(k < n2 - 1)
        def _():
            start_in(2 * k + 2, xa, ya, za, ia, sem_in_a)
        start_add(vb, ib, sem_add_b)
        return carry

    lax.fori_loop(0, n2, body, None)
    wait_add(vb, ib, sem_add_b)

    plsc.subcore_barrier()

    @pl.when(sid == 0)
    def _():
        pltpu.sync_copy(acc, out_hbm.at[cid])


def kernel(neighbor_vectors, first_atom, n_atoms):
    del n_atoms
    n_edges = first_atom.shape[0]
    assert n_edges % (NW * 2 * B) == 0

    xs = neighbor_vectors[:, 0]
    ys = neighbor_vectors[:, 1]
    zs = neighbor_vectors[:, 2]
    zero_rows = jnp.zeros((ROWS_PER_TILE, ROW_PAD), jnp.float32)

    mesh = plsc.VectorSubcoreMesh(
        core_axis_name="c", subcore_axis_name="s", num_cores=NC,
        num_subcores=NS)
    partial = pl.kernel(
        _cv_kernel,
        out_type=jax.ShapeDtypeStruct((NC, ACC_ROWS, ROW_PAD), jnp.float32),
        mesh=mesh,
        scratch_types=[
            pltpu.VMEM((B,), jnp.float32),
            pltpu.VMEM((B,), jnp.float32),
            pltpu.VMEM((B,), jnp.float32),
            pltpu.VMEM((B,), jnp.int32),
            pltpu.VMEM((B, ROW_PAD), jnp.float32),
            pltpu.VMEM((B,), jnp.float32),
            pltpu.VMEM((B,), jnp.float32),
            pltpu.VMEM((B,), jnp.float32),
            pltpu.VMEM((B,), jnp.int32),
            pltpu.VMEM((B, ROW_PAD), jnp.float32),
            pltpu.VMEM_SHARED((ACC_ROWS, ROW_PAD), jnp.float32),
            pltpu.SemaphoreType.DMA,
            pltpu.SemaphoreType.DMA,
            pltpu.SemaphoreType.DMA,
            pltpu.SemaphoreType.DMA,
        ],
        compiler_params=pltpu.CompilerParams(
            needs_layout_passes=False, use_tc_tiling_on_sc=False),
    )(xs, ys, zs, first_atom, zero_rows)
    return (partial[0] + partial[1])[:, :2]
